# Initial kernel scaffold; baseline (speedup 1.0000x reference)
import jax
import jax.numpy as jnp
from jax.experimental import pallas as pl


def kernel(x_hr, edge_index, edge_type, node_type, W):
    raise NotImplementedError("write your pallas kernel here")


# speedup vs baseline: 5.9321x; 5.9321x over previous
import functools

import jax
import jax.numpy as jnp
from jax import lax
from jax.experimental import pallas as pl
from jax.experimental.pallas import tpu as pltpu
from jax.experimental.pallas import tpu_sc as plsc

_N = 100000
_E = 1600000
_IN_C = 32
_OUT_C = 64
_NET = 7
_AUG = 37
_YC = _NET * _OUT_C
_WROWS = 40

_ROW_BLK = 512
_GRID = (_N + _ROW_BLK - 1) // _ROW_BLK

_NTILE = 16
_LN = 128
_CHUNKS = 8
_ROWS_PT = 784
_EPT = _ROWS_PT * _LN
_BLOCKS = _ROWS_PT // _CHUNKS
_E_PAD = _EPT * _NTILE
_ER = _E_PAD // _LN
_ACC_ROWS = _N + 8
_OPT = _N // _NTILE
_ZR = 625


def _y_body(x_ref, nt_ref, w_ref, y_ref):
    xb = x_ref[...]
    ntb = nt_ref[...]
    w1 = w_ref[:_IN_C, :]
    w2 = w_ref[_IN_C:, :]
    oh = (lax.broadcasted_iota(jnp.int32, (_ROW_BLK, _WROWS - _IN_C), 1)
          == ntb).astype(jnp.float32)
    acc = jnp.dot(xb, w1, preferred_element_type=jnp.float32)
    acc = acc + jnp.dot(oh, w2, preferred_element_type=jnp.float32)
    y_ref[...] = acc


_y_call = pl.pallas_call(
    _y_body,
    grid=(_GRID,),
    in_specs=[
        pl.BlockSpec((_ROW_BLK, _IN_C), lambda i: (i, 0)),
        pl.BlockSpec((_ROW_BLK, 1), lambda i: (i, 0)),
        pl.BlockSpec((_WROWS, _YC), lambda i: (0, 0)),
    ],
    out_specs=pl.BlockSpec((_ROW_BLK, _YC), lambda i: (i, 0)),
    out_shape=jax.ShapeDtypeStruct((_N, _YC), jnp.float32),
)


@functools.partial(
    pl.kernel,
    out_type=jax.ShapeDtypeStruct((_N, _OUT_C), jnp.float32),
    mesh=plsc.VectorSubcoreMesh(core_axis_name="c", subcore_axis_name="s"),
    compiler_params=pltpu.CompilerParams(use_tc_tiling_on_sc=False),
    scratch_types=[
        pltpu.VMEM((_CHUNKS, _LN), jnp.int32),
        pltpu.VMEM((_CHUNKS, _LN), jnp.int32),
        pltpu.VMEM((_CHUNKS, _LN), jnp.int32),
        pltpu.VMEM((_CHUNKS, _LN), jnp.int32),
        pltpu.VMEM((_CHUNKS, _LN, 16), jnp.float32),
        pltpu.VMEM((_ZR, 16), jnp.float32),
        pltpu.VMEM_SHARED((_ACC_ROWS, 16), jnp.float32),
        pltpu.SemaphoreType.DMA,
    ],
)
def _sc_call(y4, rowm, colm, typm, out, row_v, col_v, typ_v, idx_v, ybuf,
             zbuf, acc, sem):
    c = lax.axis_index("c")
    s = lax.axis_index("s")

    def zfill(i, carry):
        zbuf[i, :] = jnp.zeros((16,), jnp.float32)
        return carry

    lax.fori_loop(0, _ZR, zfill, 0)

    for fb in range(4):
        @pl.when(c == fb // 2)
        def _pass(fb=fb):
            def zcp(z, carry):
                pltpu.sync_copy(zbuf, acc.at[pl.ds(s * _OPT + z * _ZR, _ZR), :])
                return carry

            lax.fori_loop(0, _OPT // _ZR, zcp, 0)
            plsc.subcore_barrier()

            base = s * _ROWS_PT

            def blk_body(b, carry):
                r0 = base + b * _CHUNKS
                pltpu.sync_copy(rowm.at[pl.ds(r0, _CHUNKS), :], row_v)
                pltpu.sync_copy(colm.at[pl.ds(r0, _CHUNKS), :], col_v)
                pltpu.sync_copy(typm.at[pl.ds(r0, _CHUNKS), :], typ_v)
                for j in range(_CHUNKS):
                    for k in range(_LN // 16):
                        sl = (j, pl.ds(k * 16, 16))
                        idx_v[sl] = col_v[sl] * 28 + typ_v[sl] * 4 + fb
                handles = [
                    pltpu.async_copy(y4.at[idx_v.at[j]], ybuf.at[j], sem)
                    for j in range(_CHUNKS)
                ]
                for h in handles:
                    h.wait()
                for j in range(_CHUNKS):
                    pltpu.sync_copy(ybuf.at[j], acc.at[row_v.at[j]], add=True)
                return carry

            lax.fori_loop(0, _BLOCKS, blk_body, 0)
            plsc.subcore_barrier()
            pltpu.sync_copy(
                acc.at[pl.ds(s * _OPT, _OPT), :],
                out.at[pl.ds(s * _OPT, _OPT), pl.ds(16 * fb, 16)],
            )
            plsc.subcore_barrier()


def kernel(x_hr, edge_index, edge_type, node_type, W):
    wt = W.reshape(_NET, _AUG, _OUT_C).transpose(1, 0, 2).reshape(_AUG, _YC)
    wp = jnp.zeros((_WROWS, _YC), jnp.float32).at[:_AUG].set(wt / 7.0)
    nt2 = node_type.reshape(_N, 1)
    y = _y_call(x_hr, nt2, wp)
    y4 = y.reshape(_N * 28, 16)
    pad = _E_PAD - _E
    row = jnp.concatenate(
        [edge_index[0], jnp.full((pad,), _N, jnp.int32)]).reshape(_ER, _LN)
    col = jnp.concatenate(
        [edge_index[1], jnp.zeros((pad,), jnp.int32)]).reshape(_ER, _LN)
    typ = jnp.concatenate(
        [edge_type, jnp.zeros((pad,), jnp.int32)]).reshape(_ER, _LN)
    return _sc_call(y4, row, col, typ)


# speedup vs baseline: 6.6359x; 1.1186x over previous
import functools

import jax
import jax.numpy as jnp
from jax import lax
from jax.experimental import pallas as pl
from jax.experimental.pallas import tpu as pltpu
from jax.experimental.pallas import tpu_sc as plsc

_N = 100000
_E = 1600000
_IN_C = 32
_OUT_C = 64
_NET = 7
_AUG = 37
_YC = _NET * _OUT_C
_WROWS = 40

_ROW_BLK = 512
_GRID = (_N + _ROW_BLK - 1) // _ROW_BLK

_NTILE = 16
_LN = 128
_CHUNKS = 4
_ROWS_PT = 784
_EPT = _ROWS_PT * _LN
_BLOCKS = _ROWS_PT // _CHUNKS
_PAIRS = _BLOCKS // 2
_E_PAD = _EPT * _NTILE
_ER = _E_PAD // _LN
_ACC_ROWS = _N + 8
_OPT = _N // _NTILE
_ZR = 125


def _y_body(x_ref, nt_ref, w_ref, y_ref):
    xb = x_ref[...]
    ntb = nt_ref[...]
    w1 = w_ref[:_IN_C, :]
    w2 = w_ref[_IN_C:, :]
    oh = (lax.broadcasted_iota(jnp.int32, (_ROW_BLK, _WROWS - _IN_C), 1)
          == ntb).astype(jnp.float32)
    acc = jnp.dot(xb, w1, preferred_element_type=jnp.float32)
    acc = acc + jnp.dot(oh, w2, preferred_element_type=jnp.float32)
    y_ref[...] = acc


_y_call = pl.pallas_call(
    _y_body,
    grid=(_GRID,),
    in_specs=[
        pl.BlockSpec((_ROW_BLK, _IN_C), lambda i: (i, 0)),
        pl.BlockSpec((_ROW_BLK, 1), lambda i: (i, 0)),
        pl.BlockSpec((_WROWS, _YC), lambda i: (0, 0)),
    ],
    out_specs=pl.BlockSpec((_ROW_BLK, _YC), lambda i: (i, 0)),
    out_shape=jax.ShapeDtypeStruct((_N, _YC), jnp.float32),
)


@functools.partial(
    pl.kernel,
    out_type=jax.ShapeDtypeStruct((_N, _OUT_C), jnp.float32),
    mesh=plsc.VectorSubcoreMesh(core_axis_name="c", subcore_axis_name="s"),
    compiler_params=pltpu.CompilerParams(use_tc_tiling_on_sc=False),
    scratch_types=[
        pltpu.VMEM((2, _CHUNKS, 3, _LN), jnp.int32),
        pltpu.VMEM((2, _CHUNKS, _LN), jnp.int32),
        pltpu.VMEM((2, _CHUNKS, _LN, 16), jnp.float32),
        pltpu.VMEM((_ZR, 16), jnp.float32),
        pltpu.VMEM_SHARED((_ACC_ROWS, 16), jnp.float32),
        pltpu.SemaphoreType.DMA,
        pltpu.SemaphoreType.DMA,
    ],
)
def _sc_call(y4, packed, out, edge_v, idx_v, ybuf, zbuf, acc, sem_e, sem_g):
    c = lax.axis_index("c")
    s = lax.axis_index("s")

    def zfill(i, carry):
        zbuf[i, :] = jnp.zeros((16,), jnp.float32)
        return carry

    lax.fori_loop(0, _ZR, zfill, 0)

    def edges_start(r0, p):
        return pltpu.async_copy(packed.at[pl.ds(r0, _CHUNKS)], edge_v.at[p],
                                sem_e)

    def edges_wait(p):
        pltpu.make_async_copy(packed.at[pl.ds(0, _CHUNKS)], edge_v.at[p],
                              sem_e).wait()

    def idx_compute(p, fb):
        for j in range(_CHUNKS):
            for k in range(_LN // 16):
                sl = pl.ds(k * 16, 16)
                idx_v[p, j, sl] = (edge_v[p, j, 1, sl] * 28
                                   + edge_v[p, j, 2, sl] * 4 + fb)

    def gathers_start(p):
        for j in range(_CHUNKS):
            pltpu.async_copy(y4.at[idx_v.at[p, j]], ybuf.at[p, j], sem_g)

    def gathers_wait(p):
        for j in range(_CHUNKS):
            pltpu.make_async_copy(y4.at[idx_v.at[p, j]], ybuf.at[p, j],
                                  sem_g).wait()

    def scatters(p):
        for j in range(_CHUNKS):
            pltpu.sync_copy(ybuf.at[p, j], acc.at[edge_v.at[p, j, 0]],
                            add=True)

    for fb in range(4):
        @pl.when(c == fb // 2)
        def _pass(fb=fb):
            def zcp(z, carry):
                pltpu.sync_copy(zbuf, acc.at[pl.ds(s * _OPT + z * _ZR, _ZR), :])
                return carry

            lax.fori_loop(0, _OPT // _ZR, zcp, 0)
            plsc.subcore_barrier()

            base = s * _ROWS_PT

            for j in range(_CHUNKS):
                for k in range(_LN // 16):
                    sl = pl.ds(k * 16, 16)
                    idx_v[1, j, sl] = jnp.zeros((16,), jnp.int32)
                    edge_v[1, j, 0, sl] = jnp.full((16,), _N, jnp.int32)
            gathers_start(1)
            edges_start(base, 0)

            def pair_body(g, carry):
                b0 = base + 2 * g * _CHUNKS
                edges_wait(0)
                idx_compute(0, fb)
                gathers_wait(1)
                gathers_start(0)
                scatters(1)
                edges_start(b0 + _CHUNKS, 1)
                edges_wait(1)
                idx_compute(1, fb)
                gathers_wait(0)
                gathers_start(1)
                scatters(0)
                @pl.when(g < _PAIRS - 1)
                def _():
                    edges_start(b0 + 2 * _CHUNKS, 0)
                return carry

            lax.fori_loop(0, _PAIRS, pair_body, 0)
            gathers_wait(1)
            scatters(1)
            plsc.subcore_barrier()
            pltpu.sync_copy(
                acc.at[pl.ds(s * _OPT, _OPT), :],
                out.at[pl.ds(s * _OPT, _OPT), pl.ds(16 * fb, 16)],
            )
            plsc.subcore_barrier()


def kernel(x_hr, edge_index, edge_type, node_type, W):
    wt = W.reshape(_NET, _AUG, _OUT_C).transpose(1, 0, 2).reshape(_AUG, _YC)
    wp = jnp.zeros((_WROWS, _YC), jnp.float32).at[:_AUG].set(wt / 7.0)
    nt2 = node_type.reshape(_N, 1)
    y = _y_call(x_hr, nt2, wp)
    y4 = y.reshape(_N * 28, 16)
    pad = _E_PAD - _E
    row = jnp.concatenate([edge_index[0], jnp.full((pad,), _N, jnp.int32)])
    col = jnp.concatenate([edge_index[1], jnp.zeros((pad,), jnp.int32)])
    typ = jnp.concatenate([edge_type, jnp.zeros((pad,), jnp.int32)])
    packed = jnp.stack(
        [row.reshape(_ER, _LN), col.reshape(_ER, _LN), typ.reshape(_ER, _LN)],
        axis=1)
    return _sc_call(y4, packed)


# speedup vs baseline: 6.8612x; 1.0340x over previous
import functools

import jax
import jax.numpy as jnp
from jax import lax
from jax.experimental import pallas as pl
from jax.experimental.pallas import tpu as pltpu
from jax.experimental.pallas import tpu_sc as plsc

_N = 100000
_E = 1600000
_IN_C = 32
_OUT_C = 64
_NET = 7
_AUG = 37
_YC = _NET * _OUT_C
_WROWS = 40

_ROW_BLK = 512
_GRID = (_N + _ROW_BLK - 1) // _ROW_BLK

_NTILE = 16
_LN = 128
_BLK = 512
_EPT = 100352
_BLOCKS = _EPT // _BLK
_PAIRS = _BLOCKS // 2
_E_PAD = _EPT * _NTILE
_ACC_ROWS = _N + 8
_OPT = _N // _NTILE
_ZR = 125


def _y_body(x_ref, nt_ref, w_ref, y_ref):
    xb = x_ref[...]
    ntb = nt_ref[...]
    w1 = w_ref[:_IN_C, :]
    w2 = w_ref[_IN_C:, :]
    oh = (lax.broadcasted_iota(jnp.int32, (_ROW_BLK, _WROWS - _IN_C), 1)
          == ntb).astype(jnp.float32)
    acc = jnp.dot(xb, w1, preferred_element_type=jnp.float32)
    acc = acc + jnp.dot(oh, w2, preferred_element_type=jnp.float32)
    y_ref[...] = acc


_y_call = pl.pallas_call(
    _y_body,
    grid=(_GRID,),
    in_specs=[
        pl.BlockSpec((_ROW_BLK, _IN_C), lambda i: (i, 0)),
        pl.BlockSpec((_ROW_BLK, 1), lambda i: (i, 0)),
        pl.BlockSpec((_WROWS, _YC), lambda i: (0, 0)),
    ],
    out_specs=pl.BlockSpec((_ROW_BLK, _YC), lambda i: (i, 0)),
    out_shape=jax.ShapeDtypeStruct((_N, _YC), jnp.float32),
)


@functools.partial(
    pl.kernel,
    out_type=jax.ShapeDtypeStruct((_N, _OUT_C), jnp.float32),
    mesh=plsc.VectorSubcoreMesh(core_axis_name="c", subcore_axis_name="s"),
    compiler_params=pltpu.CompilerParams(use_tc_tiling_on_sc=False),
    scratch_types=[
        pltpu.VMEM((2, 1, _BLK), jnp.int32),
        pltpu.VMEM((2, 1, _BLK), jnp.int32),
        pltpu.VMEM((2, 1, _BLK), jnp.int32),
        pltpu.VMEM((2, 1, _BLK), jnp.int32),
        pltpu.VMEM((2, 1, _BLK, 16), jnp.float32),
        pltpu.VMEM((_ZR, 16), jnp.float32),
        pltpu.VMEM_SHARED((_ACC_ROWS, 16), jnp.float32),
        pltpu.SemaphoreType.DMA,
        pltpu.SemaphoreType.DMA,
    ],
)
def _sc_call(y4, rowm, colm, typm, out, row_v, col_v, typ_v, idx_v, ybuf,
             zbuf, acc, sem_e, sem_g):
    c = lax.axis_index("c")
    s = lax.axis_index("s")

    def zfill(i, carry):
        zbuf[i, :] = jnp.zeros((16,), jnp.float32)
        return carry

    lax.fori_loop(0, _ZR, zfill, 0)

    def edges_start(e0, p):
        pltpu.async_copy(rowm.at[pl.ds(e0, _BLK)], row_v.at[p, 0], sem_e)
        pltpu.async_copy(colm.at[pl.ds(e0, _BLK)], col_v.at[p, 0], sem_e)
        pltpu.async_copy(typm.at[pl.ds(e0, _BLK)], typ_v.at[p, 0], sem_e)

    def edges_wait(p):
        pltpu.make_async_copy(rowm.at[pl.ds(0, _BLK)], row_v.at[p, 0],
                              sem_e).wait()
        pltpu.make_async_copy(colm.at[pl.ds(0, _BLK)], col_v.at[p, 0],
                              sem_e).wait()
        pltpu.make_async_copy(typm.at[pl.ds(0, _BLK)], typ_v.at[p, 0],
                              sem_e).wait()

    def idx_compute(p, fb):
        for k in range(_BLK // 16):
            sl = pl.ds(k * 16, 16)
            idx_v[p, 0, sl] = (col_v[p, 0, sl] * 28
                               + typ_v[p, 0, sl] * 4 + fb)

    def gathers_start(p):
        pltpu.async_copy(y4.at[idx_v.at[p, 0]], ybuf.at[p, 0], sem_g)

    def gathers_wait(p):
        pltpu.make_async_copy(y4.at[idx_v.at[p, 0]], ybuf.at[p, 0],
                              sem_g).wait()

    def scatters(p):
        pltpu.sync_copy(ybuf.at[p, 0], acc.at[row_v.at[p, 0]], add=True)

    for fb in range(4):
        @pl.when(c == fb // 2)
        def _pass(fb=fb):
            def zcp(z, carry):
                pltpu.sync_copy(zbuf, acc.at[pl.ds(s * _OPT + z * _ZR, _ZR), :])
                return carry

            lax.fori_loop(0, _OPT // _ZR, zcp, 0)
            plsc.subcore_barrier()

            base = s * _EPT

            for k in range(_BLK // 16):
                sl = pl.ds(k * 16, 16)
                idx_v[1, 0, sl] = jnp.zeros((16,), jnp.int32)
                row_v[1, 0, sl] = jnp.full((16,), _N, jnp.int32)
            gathers_start(1)
            edges_start(base, 0)

            def pair_body(g, carry):
                b0 = base + 2 * g * _BLK
                edges_wait(0)
                idx_compute(0, fb)
                gathers_wait(1)
                gathers_start(0)
                scatters(1)
                edges_start(b0 + _BLK, 1)
                edges_wait(1)
                idx_compute(1, fb)
                gathers_wait(0)
                gathers_start(1)
                scatters(0)
                @pl.when(g < _PAIRS - 1)
                def _():
                    edges_start(b0 + 2 * _BLK, 0)
                return carry

            lax.fori_loop(0, _PAIRS, pair_body, 0)
            gathers_wait(1)
            scatters(1)
            plsc.subcore_barrier()
            pltpu.sync_copy(
                acc.at[pl.ds(s * _OPT, _OPT), :],
                out.at[pl.ds(s * _OPT, _OPT), pl.ds(16 * fb, 16)],
            )
            plsc.subcore_barrier()


def kernel(x_hr, edge_index, edge_type, node_type, W):
    wt = W.reshape(_NET, _AUG, _OUT_C).transpose(1, 0, 2).reshape(_AUG, _YC)
    wp = jnp.zeros((_WROWS, _YC), jnp.float32).at[:_AUG].set(wt / 7.0)
    nt2 = node_type.reshape(_N, 1)
    y = _y_call(x_hr, nt2, wp)
    y4 = y.reshape(_N * 28, 16)
    pad = _E_PAD - _E
    row = jnp.concatenate([edge_index[0], jnp.full((pad,), _N, jnp.int32)])
    col = jnp.concatenate([edge_index[1], jnp.zeros((pad,), jnp.int32)])
    typ = jnp.concatenate([edge_type, jnp.zeros((pad,), jnp.int32)])
    return _sc_call(y4, row, col, typ)


# speedup vs baseline: 8.9843x; 1.3094x over previous
import functools

import jax
import jax.numpy as jnp
from jax import lax
from jax.experimental import pallas as pl
from jax.experimental.pallas import tpu as pltpu
from jax.experimental.pallas import tpu_sc as plsc

_N = 100000
_E = 1600000
_IN_C = 32
_OUT_C = 64
_NET = 7
_AUG = 37
_YC = _NET * _OUT_C
_WROWS = 40

_ROW_BLK = 512
_GRID = (_N + _ROW_BLK - 1) // _ROW_BLK

_NTILE = 16
_BLK = 400
_EPT = _E // _NTILE
_BLOCKS = _EPT // _BLK
_PAIRS = _BLOCKS // 2
_ACC_ROWS = _N + 8
_OPT = _N // _NTILE
_ZR = 125


def _y_body(x_ref, nt_ref, w_ref, y_ref):
    xb = x_ref[...]
    ntb = nt_ref[...]
    w1 = w_ref[:_IN_C, :]
    w2 = w_ref[_IN_C:, :]
    oh = (lax.broadcasted_iota(jnp.int32, (_ROW_BLK, _WROWS - _IN_C), 1)
          == ntb).astype(jnp.float32)
    acc = jnp.dot(xb, w1, preferred_element_type=jnp.float32)
    acc = acc + jnp.dot(oh, w2, preferred_element_type=jnp.float32)
    y_ref[...] = acc.astype(jnp.bfloat16)


_y_call = pl.pallas_call(
    _y_body,
    grid=(_GRID,),
    in_specs=[
        pl.BlockSpec((_ROW_BLK, _IN_C), lambda i: (i, 0)),
        pl.BlockSpec((_ROW_BLK, 1), lambda i: (i, 0)),
        pl.BlockSpec((_WROWS, _YC), lambda i: (0, 0)),
    ],
    out_specs=pl.BlockSpec((_ROW_BLK, _YC), lambda i: (i, 0)),
    out_shape=jax.ShapeDtypeStruct((_N, _YC), jnp.bfloat16),
)


@functools.partial(
    pl.kernel,
    out_type=jax.ShapeDtypeStruct((_N, _OUT_C), jnp.bfloat16),
    mesh=plsc.VectorSubcoreMesh(core_axis_name="c", subcore_axis_name="s"),
    compiler_params=pltpu.CompilerParams(use_tc_tiling_on_sc=False),
    scratch_types=[
        pltpu.VMEM((2, 1, _BLK), jnp.int32),
        pltpu.VMEM((2, 1, _BLK), jnp.int32),
        pltpu.VMEM((2, 1, _BLK), jnp.int32),
        pltpu.VMEM((2, 1, _BLK), jnp.int32),
        pltpu.VMEM((2, 1, _BLK, 32), jnp.bfloat16),
        pltpu.VMEM((_ZR, 32), jnp.bfloat16),
        pltpu.VMEM_SHARED((_ACC_ROWS, 32), jnp.bfloat16),
        pltpu.SemaphoreType.DMA,
        pltpu.SemaphoreType.DMA,
    ],
)
def _sc_call(y2, rowm, colm, typm, out, row_v, col_v, typ_v, idx_v, ybuf,
             zbuf, acc, sem_e, sem_g):
    c = lax.axis_index("c")
    s = lax.axis_index("s")

    def zfill(i, carry):
        zbuf[i, :] = jnp.zeros((32,), jnp.bfloat16)
        return carry

    lax.fori_loop(0, _ZR, zfill, 0)

    def edges_start(e0, p):
        pltpu.async_copy(rowm.at[pl.ds(e0, _BLK)], row_v.at[p, 0], sem_e)
        pltpu.async_copy(colm.at[pl.ds(e0, _BLK)], col_v.at[p, 0], sem_e)
        pltpu.async_copy(typm.at[pl.ds(e0, _BLK)], typ_v.at[p, 0], sem_e)

    def edges_wait(p):
        pltpu.make_async_copy(rowm.at[pl.ds(0, _BLK)], row_v.at[p, 0],
                              sem_e).wait()
        pltpu.make_async_copy(colm.at[pl.ds(0, _BLK)], col_v.at[p, 0],
                              sem_e).wait()
        pltpu.make_async_copy(typm.at[pl.ds(0, _BLK)], typ_v.at[p, 0],
                              sem_e).wait()

    def idx_compute(p):
        for k in range(_BLK // 16):
            sl = pl.ds(k * 16, 16)
            idx_v[p, 0, sl] = (col_v[p, 0, sl] * 14
                               + typ_v[p, 0, sl] * 2 + c)

    def gather_start(p):
        pltpu.async_copy(y2.at[idx_v.at[p, 0]], ybuf.at[p, 0], sem_g)

    def gather_wait(p):
        pltpu.make_async_copy(y2.at[idx_v.at[p, 0]], ybuf.at[p, 0],
                              sem_g).wait()

    def scatter(p):
        pltpu.sync_copy(ybuf.at[p, 0], acc.at[row_v.at[p, 0]], add=True)

    def zcp(z, carry):
        pltpu.sync_copy(zbuf, acc.at[pl.ds(s * _OPT + z * _ZR, _ZR), :])
        return carry

    lax.fori_loop(0, _OPT // _ZR, zcp, 0)
    plsc.subcore_barrier()

    base = s * _EPT

    for k in range(_BLK // 16):
        sl = pl.ds(k * 16, 16)
        idx_v[1, 0, sl] = jnp.zeros((16,), jnp.int32)
        row_v[1, 0, sl] = jnp.full((16,), _N, jnp.int32)
    gather_start(1)
    edges_start(base, 0)

    def pair_body(g, carry):
        b0 = base + 2 * g * _BLK
        edges_wait(0)
        idx_compute(0)
        gather_wait(1)
        gather_start(0)
        scatter(1)
        edges_start(b0 + _BLK, 1)
        edges_wait(1)
        idx_compute(1)
        gather_wait(0)
        gather_start(1)
        scatter(0)
        @pl.when(g < _PAIRS - 1)
        def _():
            edges_start(b0 + 2 * _BLK, 0)
        return carry

    lax.fori_loop(0, _PAIRS, pair_body, 0)
    gather_wait(1)
    scatter(1)
    plsc.subcore_barrier()

    @pl.when(c == 0)
    def _():
        pltpu.sync_copy(acc.at[pl.ds(s * _OPT, _OPT), :],
                        out.at[pl.ds(s * _OPT, _OPT), pl.ds(0, 32)])

    @pl.when(c == 1)
    def _():
        pltpu.sync_copy(acc.at[pl.ds(s * _OPT, _OPT), :],
                        out.at[pl.ds(s * _OPT, _OPT), pl.ds(32, 32)])


def kernel(x_hr, edge_index, edge_type, node_type, W):
    wt = W.reshape(_NET, _AUG, _OUT_C).transpose(1, 0, 2).reshape(_AUG, _YC)
    wp = jnp.zeros((_WROWS, _YC), jnp.float32).at[:_AUG].set(wt / 7.0)
    nt2 = node_type.reshape(_N, 1)
    y = _y_call(x_hr, nt2, wp)
    y2 = y.reshape(_N * 14, 32)
    out_bf = _sc_call(y2, edge_index[0], edge_index[1], edge_type)
    return out_bf.astype(jnp.float32)


# speedup vs baseline: 9.7266x; 1.0826x over previous
import functools

import jax
import jax.numpy as jnp
from jax import lax
from jax.experimental import pallas as pl
from jax.experimental.pallas import tpu as pltpu
from jax.experimental.pallas import tpu_sc as plsc

_N = 100000
_E = 1600000
_IN_C = 32
_OUT_C = 64
_NET = 7
_AUG = 37
_YC = _NET * _OUT_C
_WROWS = 40

_ROW_BLK = 1024
_GRID = (_N + _ROW_BLK - 1) // _ROW_BLK

_NTILE = 16
_BLK = 400
_EPT = _E // _NTILE
_BLOCKS = _EPT // _BLK
_PAIRS = _BLOCKS // 2
_ACC_ROWS = _N + 8
_OPT = _N // _NTILE
_ZR = 125


def _y_body(x_ref, nt_ref, w_ref, y_ref):
    xb = x_ref[...]
    ntb = nt_ref[...]
    w1 = w_ref[:_IN_C, :]
    w2 = w_ref[_IN_C:, :]
    oh = (lax.broadcasted_iota(jnp.int32, (_ROW_BLK, _WROWS - _IN_C), 1)
          == ntb).astype(jnp.float32)
    acc = jnp.dot(xb, w1, preferred_element_type=jnp.float32)
    acc = acc + jnp.dot(oh, w2, preferred_element_type=jnp.float32)
    y_ref[...] = acc.astype(jnp.bfloat16)


_y_call = pl.pallas_call(
    _y_body,
    grid=(_GRID,),
    in_specs=[
        pl.BlockSpec((_ROW_BLK, _IN_C), lambda i: (i, 0)),
        pl.BlockSpec((_ROW_BLK, 1), lambda i: (i, 0)),
        pl.BlockSpec((_WROWS, _YC), lambda i: (0, 0)),
    ],
    out_specs=pl.BlockSpec((_ROW_BLK, _YC), lambda i: (i, 0)),
    out_shape=jax.ShapeDtypeStruct((_N, _YC), jnp.bfloat16),
)


@functools.partial(
    pl.kernel,
    out_type=jax.ShapeDtypeStruct((_N, _OUT_C), jnp.float32),
    mesh=plsc.VectorSubcoreMesh(core_axis_name="c", subcore_axis_name="s"),
    compiler_params=pltpu.CompilerParams(use_tc_tiling_on_sc=False,
                                         needs_layout_passes=False),
    scratch_types=[
        pltpu.VMEM((2, 1, _BLK), jnp.int32),
        pltpu.VMEM((2, 1, _BLK), jnp.int32),
        pltpu.VMEM((2, 1, _BLK), jnp.int32),
        pltpu.VMEM((2, 1, _BLK), jnp.int32),
        pltpu.VMEM((2, 1, _BLK, 32), jnp.bfloat16),
        pltpu.VMEM((_ZR, 32), jnp.bfloat16),
        pltpu.VMEM((_ZR, 32), jnp.bfloat16),
        pltpu.VMEM((_ZR, 32), jnp.float32),
        pltpu.VMEM_SHARED((_ACC_ROWS, 32), jnp.bfloat16),
        pltpu.SemaphoreType.DMA,
        pltpu.SemaphoreType.DMA,
    ],
)
def _sc_call(y2, rowm, colm, typm, out, row_v, col_v, typ_v, idx_v, ybuf,
             zbuf, cbuf, fbuf, acc, sem_e, sem_g):
    c = lax.axis_index("c")
    s = lax.axis_index("s")

    def zfill(i, carry):
        zbuf[i, :] = jnp.zeros((32,), jnp.bfloat16)
        return carry

    lax.fori_loop(0, _ZR, zfill, 0)

    def edges_start(e0, p):
        pltpu.async_copy(rowm.at[pl.ds(e0, _BLK)], row_v.at[p, 0], sem_e)
        pltpu.async_copy(colm.at[pl.ds(e0, _BLK)], col_v.at[p, 0], sem_e)
        pltpu.async_copy(typm.at[pl.ds(e0, _BLK)], typ_v.at[p, 0], sem_e)

    def edges_wait(p):
        pltpu.make_async_copy(rowm.at[pl.ds(0, _BLK)], row_v.at[p, 0],
                              sem_e).wait()
        pltpu.make_async_copy(colm.at[pl.ds(0, _BLK)], col_v.at[p, 0],
                              sem_e).wait()
        pltpu.make_async_copy(typm.at[pl.ds(0, _BLK)], typ_v.at[p, 0],
                              sem_e).wait()

    def idx_compute(p):
        for k in range(_BLK // 16):
            sl = pl.ds(k * 16, 16)
            idx_v[p, 0, sl] = (col_v[p, 0, sl] * 14
                               + typ_v[p, 0, sl] * 2 + c)

    def gather_start(p):
        pltpu.async_copy(y2.at[idx_v.at[p, 0]], ybuf.at[p, 0], sem_g)

    def gather_wait(p):
        pltpu.make_async_copy(y2.at[idx_v.at[p, 0]], ybuf.at[p, 0],
                              sem_g).wait()

    def scatter(p):
        pltpu.sync_copy(ybuf.at[p, 0], acc.at[row_v.at[p, 0]], add=True)

    def zcp(z, carry):
        pltpu.sync_copy(zbuf, acc.at[pl.ds(s * _OPT + z * _ZR, _ZR), :])
        return carry

    lax.fori_loop(0, _OPT // _ZR, zcp, 0)
    plsc.subcore_barrier()

    base = s * _EPT

    for k in range(_BLK // 16):
        sl = pl.ds(k * 16, 16)
        idx_v[1, 0, sl] = jnp.zeros((16,), jnp.int32)
        row_v[1, 0, sl] = jnp.full((16,), _N, jnp.int32)
    gather_start(1)
    edges_start(base, 0)

    def pair_body(g, carry):
        b0 = base + 2 * g * _BLK
        edges_wait(0)
        idx_compute(0)
        gather_wait(1)
        gather_start(0)
        scatter(1)
        edges_start(b0 + _BLK, 1)
        edges_wait(1)
        idx_compute(1)
        gather_wait(0)
        gather_start(1)
        scatter(0)
        @pl.when(g < _PAIRS - 1)
        def _():
            edges_start(b0 + 2 * _BLK, 0)
        return carry

    lax.fori_loop(0, _PAIRS, pair_body, 0)
    gather_wait(1)
    scatter(1)
    plsc.subcore_barrier()

    def drain(z, carry):
        r0 = s * _OPT + z * _ZR
        pltpu.sync_copy(acc.at[pl.ds(r0, _ZR), :], cbuf)

        def widen(r, carry2):
            a, b = plsc.unpack(cbuf[r, :], format=plsc.PackFormat.INTERLEAVED)
            fbuf[r, pl.ds(0, 16)] = a
            fbuf[r, pl.ds(16, 16)] = b
            return carry2

        lax.fori_loop(0, _ZR, widen, 0)

        @pl.when(c == 0)
        def _():
            pltpu.sync_copy(fbuf, out.at[pl.ds(r0, _ZR), pl.ds(0, 32)])

        @pl.when(c == 1)
        def _():
            pltpu.sync_copy(fbuf, out.at[pl.ds(r0, _ZR), pl.ds(32, 32)])

        return carry

    lax.fori_loop(0, _OPT // _ZR, drain, 0)


def kernel(x_hr, edge_index, edge_type, node_type, W):
    wt = W.reshape(_NET, _AUG, _OUT_C).transpose(1, 0, 2).reshape(_AUG, _YC)
    wt = wt.reshape(_AUG, 14, 2, 16).transpose(0, 1, 3, 2).reshape(_AUG, _YC)
    wp = jnp.zeros((_WROWS, _YC), jnp.float32).at[:_AUG].set(wt / 7.0)
    nt2 = node_type.reshape(_N, 1)
    y = _y_call(x_hr, nt2, wp)
    y2 = y.reshape(_N * 14, 32)
    return _sc_call(y2, edge_index[0], edge_index[1], edge_type)
